# Initial kernel scaffold; baseline (speedup 1.0000x reference)
import jax
import jax.numpy as jnp
from jax.experimental import pallas as pl


def kernel(x, gate_w, up_w, down_w, shared_up_w, shared_down_w):
    raise NotImplementedError("write your pallas kernel here")


# speedup vs baseline: 1.2771x; 1.2771x over previous
import functools

import jax
import jax.numpy as jnp
from jax import lax
from jax.experimental import pallas as pl
from jax.experimental.pallas import tpu as pltpu
from jax.experimental.pallas import tpu_sc as plsc

T = 2048
H = 1024
DFF = 1408
E = 8
SDFF = 2816
TILE = 256
NT = 24
NPAD = NT * TILE
NC, NS = 2, 16
NW = NC * NS
RPW = NPAD // NW
GCH = 64
TPW = T // NW
CCH = 32


def _router_body(x_ref, gw_ref, d0_ref, d1_ref, w0_ref, w1_ref, tetv_ref):
    x = x_ref[...]
    gw = gw_ref[...]
    logits = lax.dot_general(x, gw, (((1,), (1,)), ((), ())),
                             preferred_element_type=jnp.float32)
    m = jnp.max(logits, axis=1, keepdims=True)
    ex = jnp.exp(logits - m)
    iota = lax.broadcasted_iota(jnp.int32, (T, E), 1).astype(jnp.float32)
    m1 = jnp.max(ex, axis=1, keepdims=True)
    i1 = jnp.min(jnp.where(ex == m1, iota, float(E)), axis=1, keepdims=True)
    oh1 = (iota == i1).astype(jnp.float32)
    exm = jnp.where(oh1 > 0, -1.0, ex)
    m2 = jnp.max(exm, axis=1, keepdims=True)
    i2 = jnp.min(jnp.where(exm == m2, iota, float(E)), axis=1, keepdims=True)
    oh2 = (iota == i2).astype(jnp.float32)
    denom = m1 + m2 + 1e-20
    w0_ref[...] = m1 / denom
    w1_ref[...] = m2 / denom
    A = oh1 + oh2
    c = A
    s = 1
    while s < T:
        c = c + jnp.concatenate(
            [jnp.zeros((s, E), jnp.float32), c[:T - s]], axis=0)
        s *= 2
    rexcl = c - A
    ones_col = jnp.ones((T, 1), jnp.float32)
    counts_col = lax.dot_general(A, ones_col, (((0,), (0,)), ((), ())),
                                 preferred_element_type=jnp.float32)
    counts_row = jnp.sum(A, axis=0, keepdims=True)
    cpad_col = jnp.ceil(counts_col * (1.0 / TILE)) * TILE
    cpad_row = jnp.ceil(counts_row * (1.0 / TILE)) * TILE
    ke = lax.broadcasted_iota(jnp.int32, (E, E), 0)
    ee = lax.broadcasted_iota(jnp.int32, (E, E), 1)
    mt = (ke < ee).astype(jnp.float32)
    off_row = lax.dot_general(cpad_row, mt, (((1,), (0,)), ((), ())),
                              preferred_element_type=jnp.float32)
    off_col = lax.dot_general(mt, cpad_col, (((0,), (0,)), ((), ())),
                              preferred_element_type=jnp.float32)
    total = jnp.sum(cpad_col)
    rank0 = jnp.sum(oh1 * rexcl, axis=1, keepdims=True)
    rank1 = jnp.sum(oh2 * rexcl, axis=1, keepdims=True)
    d0 = jnp.sum(oh1 * off_row, axis=1, keepdims=True) + rank0
    d1 = jnp.sum(oh2 * off_row, axis=1, keepdims=True) + rank1
    d0_ref[...] = d0.astype(jnp.int32)
    d1_ref[...] = d1.astype(jnp.int32)
    tj = lax.broadcasted_iota(jnp.int32, (1, 64), 1).astype(jnp.float32) * TILE
    cmp = (tj >= off_col).astype(jnp.float32)
    te_row = jnp.clip(jnp.sum(cmp, axis=0, keepdims=True) - 1.0,
                      0.0, float(E - 1))
    tv_row = (tj < total).astype(jnp.float32)
    pad = jnp.zeros((6, 64), jnp.float32)
    tetv_ref[...] = jnp.concatenate([te_row, tv_row, pad],
                                    axis=0).astype(jnp.int32)


def _router(x, gate_w):
    return pl.pallas_call(
        _router_body,
        out_shape=(
            jax.ShapeDtypeStruct((T, 1), jnp.int32),
            jax.ShapeDtypeStruct((T, 1), jnp.int32),
            jax.ShapeDtypeStruct((T, 1), jnp.float32),
            jax.ShapeDtypeStruct((T, 1), jnp.float32),
            jax.ShapeDtypeStruct((8, 64), jnp.int32),
        ),
    )(x, gate_w)


def _wid():
    return lax.axis_index("s") * NC + lax.axis_index("c")


@functools.cache
def _mesh():
    return plsc.VectorSubcoreMesh(core_axis_name="c", subcore_axis_name="s",
                                  num_cores=NC, num_subcores=NS)


@functools.cache
def _dispatch_build_kernel():
    return pl.kernel(
        _dispatch_build_body,
        out_type=jax.ShapeDtypeStruct((NPAD,), jnp.int32),
        mesh=_mesh(),
        compiler_params=pltpu.CompilerParams(needs_layout_passes=False),
        scratch_types=[
            pltpu.VMEM((NPAD,), jnp.int32),
            pltpu.VMEM((T,), jnp.int32),
            pltpu.VMEM((T,), jnp.int32),
        ],
    )


def _dispatch_build_body(d0_hbm, d1_hbm, src_hbm, src_v, d0_v, d1_v):
    @pl.when(_wid() == 0)
    def _():
        def zero_body(i, carry):
            src_v[pl.ds(i * 16, 16)] = jnp.zeros((16,), jnp.int32)
            return carry
        lax.fori_loop(0, NPAD // 16, zero_body, 0, unroll=8)
        pltpu.sync_copy(d0_hbm, d0_v)
        pltpu.sync_copy(d1_hbm, d1_v)

        def sc_body(i, carry):
            vals = lax.iota(jnp.int32, 16) + i * 16
            plsc.store_scatter(src_v, [d0_v[pl.ds(i * 16, 16)]], vals)
            plsc.store_scatter(src_v, [d1_v[pl.ds(i * 16, 16)]], vals)
            return carry
        lax.fori_loop(0, T // 16, sc_body, 0, unroll=8)
        pltpu.sync_copy(src_v, src_hbm)


@functools.cache
def _gather_rows_kernel():
    return pl.kernel(
        _gather_rows_body,
        out_type=jax.ShapeDtypeStruct((NPAD, H), jnp.float32),
        mesh=_mesh(),
        compiler_params=pltpu.CompilerParams(needs_layout_passes=False),
        scratch_types=[
            pltpu.VMEM((RPW,), jnp.int32),
            pltpu.VMEM((GCH, H), jnp.float32),
            pltpu.SemaphoreType.DMA,
        ],
    )


def _gather_rows_body(src_hbm, x_hbm, xs_hbm, idx_v, rows_v, sem):
    base = _wid() * RPW
    pltpu.sync_copy(src_hbm.at[pl.ds(base, RPW)], idx_v)

    def body(c, carry):
        o = c * GCH
        pltpu.async_copy(x_hbm.at[idx_v.at[pl.ds(o, GCH)]], rows_v, sem).wait()
        pltpu.sync_copy(rows_v, xs_hbm.at[pl.ds(base + o, GCH)])
        return carry
    lax.fori_loop(0, RPW // GCH, body, 0)


def _grouped_body(tetv_ref, xs_ref, up_ref, dn_ref, out_ref):
    i = pl.program_id(0)

    @pl.when(tetv_ref[1, i] == 1)
    def _():
        xb = xs_ref[...]
        h = lax.dot_general(xb, up_ref[0], (((1,), (1,)), ((), ())),
                            preferred_element_type=jnp.float32)
        g = h[:, :DFF]
        u = h[:, DFF:]
        act = g * jax.nn.sigmoid(g) * u
        out_ref[...] = lax.dot_general(act, dn_ref[0], (((1,), (1,)), ((), ())),
                                       preferred_element_type=jnp.float32)


def _grouped(tetv, xs, up_w, down_w):
    grid_spec = pltpu.PrefetchScalarGridSpec(
        num_scalar_prefetch=1,
        grid=(NT,),
        in_specs=[
            pl.BlockSpec((TILE, H), lambda i, tetv: (i, 0)),
            pl.BlockSpec((1, 2 * DFF, H), lambda i, tetv: (tetv[0, i], 0, 0)),
            pl.BlockSpec((1, H, DFF), lambda i, tetv: (tetv[0, i], 0, 0)),
        ],
        out_specs=pl.BlockSpec((TILE, H), lambda i, tetv: (i, 0)),
    )
    return pl.pallas_call(
        _grouped_body,
        grid_spec=grid_spec,
        out_shape=jax.ShapeDtypeStruct((NPAD, H), jnp.float32),
    )(tetv, xs, up_w, down_w)


@functools.cache
def _combine_gather_kernel():
    return pl.kernel(
        _combine_gather_body,
        out_type=(
            jax.ShapeDtypeStruct((T, H), jnp.float32),
            jax.ShapeDtypeStruct((T, H), jnp.float32),
        ),
        mesh=_mesh(),
        compiler_params=pltpu.CompilerParams(needs_layout_passes=False),
        scratch_types=[
            pltpu.VMEM((TPW,), jnp.int32),
            pltpu.VMEM((TPW,), jnp.int32),
            pltpu.VMEM((CCH, H), jnp.float32),
            pltpu.SemaphoreType.DMA,
        ],
    )


def _combine_gather_body(d0_hbm, d1_hbm, ys_hbm, g0_hbm, g1_hbm,
                         d0_v, d1_v, rows_v, sem):
    base = _wid() * TPW
    pltpu.sync_copy(d0_hbm.at[pl.ds(base, TPW)], d0_v)
    pltpu.sync_copy(d1_hbm.at[pl.ds(base, TPW)], d1_v)

    def body(c, carry):
        o = c * CCH
        pltpu.async_copy(ys_hbm.at[d0_v.at[pl.ds(o, CCH)]], rows_v, sem).wait()
        pltpu.sync_copy(rows_v, g0_hbm.at[pl.ds(base + o, CCH)])
        pltpu.async_copy(ys_hbm.at[d1_v.at[pl.ds(o, CCH)]], rows_v, sem).wait()
        pltpu.sync_copy(rows_v, g1_hbm.at[pl.ds(base + o, CCH)])
        return carry
    lax.fori_loop(0, TPW // CCH, body, 0)


def _shared_body(x_ref, sup_ref, sdn_ref, out_ref):
    xb = x_ref[...]
    h = lax.dot_general(xb, sup_ref[...], (((1,), (1,)), ((), ())),
                        preferred_element_type=jnp.float32)
    g = h[:, :SDFF]
    u = h[:, SDFF:]
    act = g * jax.nn.sigmoid(g) * u
    out_ref[...] = lax.dot_general(act, sdn_ref[...], (((1,), (1,)), ((), ())),
                                   preferred_element_type=jnp.float32)


def _shared(x, sup, sdn):
    SB = 256
    return pl.pallas_call(
        _shared_body,
        grid=(T // SB,),
        in_specs=[
            pl.BlockSpec((SB, H), lambda i: (i, 0)),
            pl.BlockSpec((2 * SDFF, H), lambda i: (0, 0)),
            pl.BlockSpec((H, SDFF), lambda i: (0, 0)),
        ],
        out_specs=pl.BlockSpec((SB, H), lambda i: (i, 0)),
        out_shape=jax.ShapeDtypeStruct((T, H), jnp.float32),
    )(x, sup, sdn)


def _final_body(g0_ref, g1_ref, sh_ref, w0_ref, w1_ref, out_ref):
    out_ref[...] = (w0_ref[...] * g0_ref[...] + w1_ref[...] * g1_ref[...]
                    + sh_ref[...])


def _final(g0, g1, sh, w0, w1):
    SB = 256
    return pl.pallas_call(
        _final_body,
        grid=(T // SB,),
        in_specs=[
            pl.BlockSpec((SB, H), lambda i: (i, 0)),
            pl.BlockSpec((SB, H), lambda i: (i, 0)),
            pl.BlockSpec((SB, H), lambda i: (i, 0)),
            pl.BlockSpec((SB, 1), lambda i: (i, 0)),
            pl.BlockSpec((SB, 1), lambda i: (i, 0)),
        ],
        out_specs=pl.BlockSpec((SB, H), lambda i: (i, 0)),
        out_shape=jax.ShapeDtypeStruct((T, H), jnp.float32),
    )(g0, g1, sh, w0, w1)


def kernel(x, gate_w, up_w, down_w, shared_up_w, shared_down_w):
    d0, d1, w0, w1, tetv = _router(x, gate_w)
    d0f = d0.reshape(T)
    d1f = d1.reshape(T)
    src = _dispatch_build_kernel()(d0f, d1f)
    xs = _gather_rows_kernel()(src, x)
    ys = _grouped(tetv, xs, up_w, down_w)
    g0, g1 = _combine_gather_kernel()(d0f, d1f, ys)
    sh = _shared(x, shared_up_w, shared_down_w)
    return _final(g0, g1, sh, w0, w1)


# speedup vs baseline: 1.2793x; 1.0017x over previous
import functools

import jax
import jax.numpy as jnp
from jax import lax
from jax.experimental import pallas as pl
from jax.experimental.pallas import tpu as pltpu
from jax.experimental.pallas import tpu_sc as plsc

T = 2048
H = 1024
DFF = 1408
E = 8
SDFF = 2816
TILE = 256
NT = 24
NPAD = NT * TILE
NC, NS = 2, 16
NW = NC * NS
RPW = NPAD // NW
GCH = 48
TPW = T // NW
CCH = 32


def _router_body(x_ref, gw_ref, d0_ref, d1_ref, w0_ref, w1_ref, tetv_ref):
    x = x_ref[...]
    gw = gw_ref[...]
    logits = lax.dot_general(x, gw, (((1,), (1,)), ((), ())),
                             preferred_element_type=jnp.float32)
    m = jnp.max(logits, axis=1, keepdims=True)
    ex = jnp.exp(logits - m)
    iota = lax.broadcasted_iota(jnp.int32, (T, E), 1).astype(jnp.float32)
    m1 = jnp.max(ex, axis=1, keepdims=True)
    i1 = jnp.min(jnp.where(ex == m1, iota, float(E)), axis=1, keepdims=True)
    oh1 = (iota == i1).astype(jnp.float32)
    exm = jnp.where(oh1 > 0, -1.0, ex)
    m2 = jnp.max(exm, axis=1, keepdims=True)
    i2 = jnp.min(jnp.where(exm == m2, iota, float(E)), axis=1, keepdims=True)
    oh2 = (iota == i2).astype(jnp.float32)
    denom = m1 + m2 + 1e-20
    w0_ref[...] = m1 / denom
    w1_ref[...] = m2 / denom
    A = oh1 + oh2
    c = A
    s = 1
    while s < T:
        c = c + jnp.concatenate(
            [jnp.zeros((s, E), jnp.float32), c[:T - s]], axis=0)
        s *= 2
    rexcl = c - A
    ones_col = jnp.ones((T, 1), jnp.float32)
    counts_col = lax.dot_general(A, ones_col, (((0,), (0,)), ((), ())),
                                 preferred_element_type=jnp.float32)
    counts_row = jnp.sum(A, axis=0, keepdims=True)
    cpad_col = jnp.ceil(counts_col * (1.0 / TILE)) * TILE
    cpad_row = jnp.ceil(counts_row * (1.0 / TILE)) * TILE
    ke = lax.broadcasted_iota(jnp.int32, (E, E), 0)
    ee = lax.broadcasted_iota(jnp.int32, (E, E), 1)
    mt = (ke < ee).astype(jnp.float32)
    off_row = lax.dot_general(cpad_row, mt, (((1,), (0,)), ((), ())),
                              preferred_element_type=jnp.float32)
    off_col = lax.dot_general(mt, cpad_col, (((0,), (0,)), ((), ())),
                              preferred_element_type=jnp.float32)
    total = jnp.sum(cpad_col)
    rank0 = jnp.sum(oh1 * rexcl, axis=1, keepdims=True)
    rank1 = jnp.sum(oh2 * rexcl, axis=1, keepdims=True)
    d0 = jnp.sum(oh1 * off_row, axis=1, keepdims=True) + rank0
    d1 = jnp.sum(oh2 * off_row, axis=1, keepdims=True) + rank1
    d0_ref[...] = d0.astype(jnp.int32)
    d1_ref[...] = d1.astype(jnp.int32)
    tj = lax.broadcasted_iota(jnp.int32, (1, 64), 1).astype(jnp.float32) * TILE
    cmp = (tj >= off_col).astype(jnp.float32)
    te_row = jnp.clip(jnp.sum(cmp, axis=0, keepdims=True) - 1.0,
                      0.0, float(E - 1))
    tv_row = (tj < total).astype(jnp.float32)
    pad = jnp.zeros((6, 64), jnp.float32)
    tetv_ref[...] = jnp.concatenate([te_row, tv_row, pad],
                                    axis=0).astype(jnp.int32)


def _router(x, gate_w):
    return pl.pallas_call(
        _router_body,
        out_shape=(
            jax.ShapeDtypeStruct((T, 1), jnp.int32),
            jax.ShapeDtypeStruct((T, 1), jnp.int32),
            jax.ShapeDtypeStruct((T, 1), jnp.float32),
            jax.ShapeDtypeStruct((T, 1), jnp.float32),
            jax.ShapeDtypeStruct((8, 64), jnp.int32),
        ),
    )(x, gate_w)


def _wid():
    return lax.axis_index("s") * NC + lax.axis_index("c")


@functools.cache
def _mesh():
    return plsc.VectorSubcoreMesh(core_axis_name="c", subcore_axis_name="s",
                                  num_cores=NC, num_subcores=NS)


@functools.cache
def _dispatch_build_kernel():
    return pl.kernel(
        _dispatch_build_body,
        out_type=jax.ShapeDtypeStruct((NPAD,), jnp.int32),
        mesh=_mesh(),
        compiler_params=pltpu.CompilerParams(needs_layout_passes=False),
        scratch_types=[
            pltpu.VMEM((NPAD,), jnp.int32),
            pltpu.VMEM((T,), jnp.int32),
            pltpu.VMEM((T,), jnp.int32),
        ],
    )


def _dispatch_build_body(d0_hbm, d1_hbm, src_hbm, src_v, d0_v, d1_v):
    @pl.when(_wid() == 0)
    def _():
        def zero_body(i, carry):
            src_v[pl.ds(i * 16, 16)] = jnp.zeros((16,), jnp.int32)
            return carry
        lax.fori_loop(0, NPAD // 16, zero_body, 0, unroll=8)
        pltpu.sync_copy(d0_hbm, d0_v)
        pltpu.sync_copy(d1_hbm, d1_v)

        def sc_body(i, carry):
            vals = lax.iota(jnp.int32, 16) + i * 16
            plsc.store_scatter(src_v, [d0_v[pl.ds(i * 16, 16)]], vals)
            plsc.store_scatter(src_v, [d1_v[pl.ds(i * 16, 16)]], vals)
            return carry
        lax.fori_loop(0, T // 16, sc_body, 0, unroll=8)
        pltpu.sync_copy(src_v, src_hbm)


@functools.cache
def _gather_rows_kernel():
    return pl.kernel(
        _gather_rows_body,
        out_type=jax.ShapeDtypeStruct((NPAD, H), jnp.float32),
        mesh=_mesh(),
        compiler_params=pltpu.CompilerParams(needs_layout_passes=False),
        scratch_types=[
            pltpu.VMEM((RPW,), jnp.int32),
            pltpu.VMEM((GCH, H), jnp.float32),
            pltpu.VMEM((GCH, H), jnp.float32),
            pltpu.SemaphoreType.DMA,
            pltpu.SemaphoreType.DMA,
            pltpu.SemaphoreType.DMA,
            pltpu.SemaphoreType.DMA,
        ],
    )


def _gather_rows_body(src_hbm, x_hbm, xs_hbm, idx_v, r0, r1, g0, g1, w0, w1):
    base = _wid() * RPW
    pltpu.sync_copy(src_hbm.at[pl.ds(base, RPW)], idx_v)
    bufs = (r0, r1)
    gsems = (g0, g1)
    wsems = (w0, w1)
    nch = RPW // GCH
    gcps = [None, None]
    wcps = [None, None]
    for c in range(nch):
        b = c % 2
        if wcps[b] is not None:
            wcps[b].wait()
        gcps[b] = pltpu.async_copy(
            x_hbm.at[idx_v.at[pl.ds(c * GCH, GCH)]], bufs[b], gsems[b])
        pb = 1 - b
        if c >= 1:
            gcps[pb].wait()
            wcps[pb] = pltpu.async_copy(
                bufs[pb], xs_hbm.at[pl.ds(base + (c - 1) * GCH, GCH)],
                wsems[pb])
    lb = (nch - 1) % 2
    gcps[lb].wait()
    wlast = pltpu.async_copy(
        bufs[lb], xs_hbm.at[pl.ds(base + (nch - 1) * GCH, GCH)], wsems[lb])
    if wcps[1 - lb] is not None:
        wcps[1 - lb].wait()
    wlast.wait()


def _grouped_body(tetv_ref, xs_ref, up_ref, dn_ref, out_ref):
    i = pl.program_id(0)

    @pl.when(tetv_ref[1, i] == 1)
    def _():
        xb = xs_ref[...]
        h = lax.dot_general(xb, up_ref[0], (((1,), (1,)), ((), ())),
                            preferred_element_type=jnp.float32)
        g = h[:, :DFF]
        u = h[:, DFF:]
        act = g * jax.nn.sigmoid(g) * u
        out_ref[...] = lax.dot_general(act, dn_ref[0], (((1,), (1,)), ((), ())),
                                       preferred_element_type=jnp.float32)


def _grouped(tetv, xs, up_w, down_w):
    grid_spec = pltpu.PrefetchScalarGridSpec(
        num_scalar_prefetch=1,
        grid=(NT,),
        in_specs=[
            pl.BlockSpec((TILE, H), lambda i, tetv: (i, 0)),
            pl.BlockSpec((1, 2 * DFF, H), lambda i, tetv: (tetv[0, i], 0, 0)),
            pl.BlockSpec((1, H, DFF), lambda i, tetv: (tetv[0, i], 0, 0)),
        ],
        out_specs=pl.BlockSpec((TILE, H), lambda i, tetv: (i, 0)),
    )
    return pl.pallas_call(
        _grouped_body,
        grid_spec=grid_spec,
        out_shape=jax.ShapeDtypeStruct((NPAD, H), jnp.float32),
    )(tetv, xs, up_w, down_w)


@functools.cache
def _combine_gather_kernel():
    return pl.kernel(
        _combine_gather_body,
        out_type=(
            jax.ShapeDtypeStruct((T, H), jnp.float32),
            jax.ShapeDtypeStruct((T, H), jnp.float32),
        ),
        mesh=_mesh(),
        compiler_params=pltpu.CompilerParams(needs_layout_passes=False),
        scratch_types=[
            pltpu.VMEM((TPW,), jnp.int32),
            pltpu.VMEM((TPW,), jnp.int32),
            pltpu.VMEM((CCH, H), jnp.float32),
            pltpu.SemaphoreType.DMA,
        ],
    )


def _combine_gather_body(d0_hbm, d1_hbm, ys_hbm, g0_hbm, g1_hbm,
                         d0_v, d1_v, rows_v, sem):
    base = _wid() * TPW
    pltpu.sync_copy(d0_hbm.at[pl.ds(base, TPW)], d0_v)
    pltpu.sync_copy(d1_hbm.at[pl.ds(base, TPW)], d1_v)

    def body(c, carry):
        o = c * CCH
        pltpu.async_copy(ys_hbm.at[d0_v.at[pl.ds(o, CCH)]], rows_v, sem).wait()
        pltpu.sync_copy(rows_v, g0_hbm.at[pl.ds(base + o, CCH)])
        pltpu.async_copy(ys_hbm.at[d1_v.at[pl.ds(o, CCH)]], rows_v, sem).wait()
        pltpu.sync_copy(rows_v, g1_hbm.at[pl.ds(base + o, CCH)])
        return carry
    lax.fori_loop(0, TPW // CCH, body, 0)


def _shared_body(x_ref, sup_ref, sdn_ref, out_ref):
    xb = x_ref[...]
    h = lax.dot_general(xb, sup_ref[...], (((1,), (1,)), ((), ())),
                        preferred_element_type=jnp.float32)
    g = h[:, :SDFF]
    u = h[:, SDFF:]
    act = g * jax.nn.sigmoid(g) * u
    out_ref[...] = lax.dot_general(act, sdn_ref[...], (((1,), (1,)), ((), ())),
                                   preferred_element_type=jnp.float32)


def _shared(x, sup, sdn):
    SB = 256
    return pl.pallas_call(
        _shared_body,
        grid=(T // SB,),
        in_specs=[
            pl.BlockSpec((SB, H), lambda i: (i, 0)),
            pl.BlockSpec((2 * SDFF, H), lambda i: (0, 0)),
            pl.BlockSpec((H, SDFF), lambda i: (0, 0)),
        ],
        out_specs=pl.BlockSpec((SB, H), lambda i: (i, 0)),
        out_shape=jax.ShapeDtypeStruct((T, H), jnp.float32),
    )(x, sup, sdn)


def _final_body(g0_ref, g1_ref, sh_ref, w0_ref, w1_ref, out_ref):
    out_ref[...] = (w0_ref[...] * g0_ref[...] + w1_ref[...] * g1_ref[...]
                    + sh_ref[...])


def _final(g0, g1, sh, w0, w1):
    SB = 256
    return pl.pallas_call(
        _final_body,
        grid=(T // SB,),
        in_specs=[
            pl.BlockSpec((SB, H), lambda i: (i, 0)),
            pl.BlockSpec((SB, H), lambda i: (i, 0)),
            pl.BlockSpec((SB, H), lambda i: (i, 0)),
            pl.BlockSpec((SB, 1), lambda i: (i, 0)),
            pl.BlockSpec((SB, 1), lambda i: (i, 0)),
        ],
        out_specs=pl.BlockSpec((SB, H), lambda i: (i, 0)),
        out_shape=jax.ShapeDtypeStruct((T, H), jnp.float32),
    )(g0, g1, sh, w0, w1)


def kernel(x, gate_w, up_w, down_w, shared_up_w, shared_down_w):
    d0, d1, w0, w1, tetv = _router(x, gate_w)
    d0f = d0.reshape(T)
    d1f = d1.reshape(T)
    src = _dispatch_build_kernel()(d0f, d1f)
    xs = _gather_rows_kernel()(src, x)
    sh = _shared(x, shared_up_w, shared_down_w)
    ys = _grouped(tetv, xs, up_w, down_w)
    g0, g1 = _combine_gather_kernel()(d0f, d1f, ys)
    return _final(g0, g1, sh, w0, w1)


# speedup vs baseline: 1.8671x; 1.4594x over previous
import functools

import jax
import jax.numpy as jnp
from jax import lax
from jax.experimental import pallas as pl
from jax.experimental.pallas import tpu as pltpu
from jax.experimental.pallas import tpu_sc as plsc

T = 2048
H = 1024
DFF = 1408
E = 8
SDFF = 2816
TILE = 256
NT = 24
NPAD = NT * TILE
NC, NS = 2, 16
NW = NC * NS
RPW = NPAD // NW
GCH = 48
TPW = T // NW
CCH = 32


def _router_body(x_ref, gw_ref, d0_ref, d1_ref, w0_ref, w1_ref, tetv_ref):
    x = x_ref[...]
    gw = gw_ref[...]
    logits = lax.dot_general(x, gw, (((1,), (1,)), ((), ())),
                             preferred_element_type=jnp.float32)
    m = jnp.max(logits, axis=1, keepdims=True)
    ex = jnp.exp(logits - m)
    iota = lax.broadcasted_iota(jnp.int32, (T, E), 1).astype(jnp.float32)
    m1 = jnp.max(ex, axis=1, keepdims=True)
    i1 = jnp.min(jnp.where(ex == m1, iota, float(E)), axis=1, keepdims=True)
    oh1 = (iota == i1).astype(jnp.float32)
    exm = jnp.where(oh1 > 0, -1.0, ex)
    m2 = jnp.max(exm, axis=1, keepdims=True)
    i2 = jnp.min(jnp.where(exm == m2, iota, float(E)), axis=1, keepdims=True)
    oh2 = (iota == i2).astype(jnp.float32)
    denom = m1 + m2 + 1e-20
    w0_ref[...] = m1 / denom
    w1_ref[...] = m2 / denom
    A = oh1 + oh2
    c = A
    s = 1
    while s < T:
        c = c + jnp.concatenate(
            [jnp.zeros((s, E), jnp.float32), c[:T - s]], axis=0)
        s *= 2
    rexcl = c - A
    ones_col = jnp.ones((T, 1), jnp.float32)
    counts_col = lax.dot_general(A, ones_col, (((0,), (0,)), ((), ())),
                                 preferred_element_type=jnp.float32)
    counts_row = jnp.sum(A, axis=0, keepdims=True)
    cpad_col = jnp.ceil(counts_col * (1.0 / TILE)) * TILE
    cpad_row = jnp.ceil(counts_row * (1.0 / TILE)) * TILE
    ke = lax.broadcasted_iota(jnp.int32, (E, E), 0)
    ee = lax.broadcasted_iota(jnp.int32, (E, E), 1)
    mt = (ke < ee).astype(jnp.float32)
    off_row = lax.dot_general(cpad_row, mt, (((1,), (0,)), ((), ())),
                              preferred_element_type=jnp.float32)
    off_col = lax.dot_general(mt, cpad_col, (((0,), (0,)), ((), ())),
                              preferred_element_type=jnp.float32)
    total = jnp.sum(cpad_col)
    rank0 = jnp.sum(oh1 * rexcl, axis=1, keepdims=True)
    rank1 = jnp.sum(oh2 * rexcl, axis=1, keepdims=True)
    d0 = jnp.sum(oh1 * off_row, axis=1, keepdims=True) + rank0
    d1 = jnp.sum(oh2 * off_row, axis=1, keepdims=True) + rank1
    d0_ref[...] = d0.astype(jnp.int32)
    d1_ref[...] = d1.astype(jnp.int32)
    tj = lax.broadcasted_iota(jnp.int32, (1, 64), 1).astype(jnp.float32) * TILE
    cmp = (tj >= off_col).astype(jnp.float32)
    te_row = jnp.clip(jnp.sum(cmp, axis=0, keepdims=True) - 1.0,
                      0.0, float(E - 1))
    tv_row = (tj < total).astype(jnp.float32)
    pad = jnp.zeros((6, 64), jnp.float32)
    tetv_ref[...] = jnp.concatenate([te_row, tv_row, pad],
                                    axis=0).astype(jnp.int32)


def _router(x, gate_w):
    return pl.pallas_call(
        _router_body,
        out_shape=(
            jax.ShapeDtypeStruct((T, 1), jnp.int32),
            jax.ShapeDtypeStruct((T, 1), jnp.int32),
            jax.ShapeDtypeStruct((T, 1), jnp.float32),
            jax.ShapeDtypeStruct((T, 1), jnp.float32),
            jax.ShapeDtypeStruct((8, 64), jnp.int32),
        ),
    )(x, gate_w)


def _wid():
    return lax.axis_index("s") * NC + lax.axis_index("c")


@functools.cache
def _mesh():
    return plsc.VectorSubcoreMesh(core_axis_name="c", subcore_axis_name="s",
                                  num_cores=NC, num_subcores=NS)


@functools.cache
def _dispatch_build_kernel():
    return pl.kernel(
        _dispatch_build_body,
        out_type=jax.ShapeDtypeStruct((NPAD,), jnp.int32),
        mesh=_mesh(),
        compiler_params=pltpu.CompilerParams(needs_layout_passes=False),
        scratch_types=[
            pltpu.VMEM((NPAD,), jnp.int32),
            pltpu.VMEM((T,), jnp.int32),
            pltpu.VMEM((T,), jnp.int32),
        ],
    )


def _dispatch_build_body(d0_hbm, d1_hbm, src_hbm, src_v, d0_v, d1_v):
    @pl.when(_wid() == 0)
    def _():
        def zero_body(i, carry):
            src_v[pl.ds(i * 16, 16)] = (lax.iota(jnp.int32, 16) + i * 16) & (T - 1)
            return carry
        lax.fori_loop(0, NPAD // 16, zero_body, 0, unroll=8)
        pltpu.sync_copy(d0_hbm, d0_v)
        pltpu.sync_copy(d1_hbm, d1_v)

        def sc_body(i, carry):
            vals = lax.iota(jnp.int32, 16) + i * 16
            plsc.store_scatter(src_v, [d0_v[pl.ds(i * 16, 16)]], vals)
            plsc.store_scatter(src_v, [d1_v[pl.ds(i * 16, 16)]], vals)
            return carry
        lax.fori_loop(0, T // 16, sc_body, 0, unroll=8)
        pltpu.sync_copy(src_v, src_hbm)


@functools.cache
def _gather_rows_kernel():
    return pl.kernel(
        _gather_rows_body,
        out_type=jax.ShapeDtypeStruct((NPAD, H), jnp.float32),
        mesh=_mesh(),
        compiler_params=pltpu.CompilerParams(needs_layout_passes=False),
        scratch_types=[
            pltpu.VMEM((RPW,), jnp.int32),
            pltpu.VMEM((GCH, H), jnp.float32),
            pltpu.VMEM((GCH, H), jnp.float32),
            pltpu.SemaphoreType.DMA,
            pltpu.SemaphoreType.DMA,
            pltpu.SemaphoreType.DMA,
            pltpu.SemaphoreType.DMA,
        ],
    )


def _gather_rows_body(src_hbm, x_hbm, xs_hbm, idx_v, r0, r1, g0, g1, w0, w1):
    base = _wid() * RPW
    pltpu.sync_copy(src_hbm.at[pl.ds(base, RPW)], idx_v)
    bufs = (r0, r1)
    gsems = (g0, g1)
    wsems = (w0, w1)
    nch = RPW // GCH
    gcps = [None, None]
    wcps = [None, None]
    for c in range(nch):
        b = c % 2
        if wcps[b] is not None:
            wcps[b].wait()
        gcps[b] = pltpu.async_copy(
            x_hbm.at[idx_v.at[pl.ds(c * GCH, GCH)]], bufs[b], gsems[b])
        pb = 1 - b
        if c >= 1:
            gcps[pb].wait()
            wcps[pb] = pltpu.async_copy(
                bufs[pb], xs_hbm.at[pl.ds(base + (c - 1) * GCH, GCH)],
                wsems[pb])
    lb = (nch - 1) % 2
    gcps[lb].wait()
    wlast = pltpu.async_copy(
        bufs[lb], xs_hbm.at[pl.ds(base + (nch - 1) * GCH, GCH)], wsems[lb])
    if wcps[1 - lb] is not None:
        wcps[1 - lb].wait()
    wlast.wait()


def _grouped_body(tetv_ref, xs_ref, up_ref, dn_ref, out_ref):
    i = pl.program_id(0)

    @pl.when(tetv_ref[1, i] == 1)
    def _():
        xb = xs_ref[...]
        h = lax.dot_general(xb, up_ref[0], (((1,), (1,)), ((), ())),
                            preferred_element_type=jnp.float32)
        g = h[:, :DFF]
        u = h[:, DFF:]
        act = g * jax.nn.sigmoid(g) * u
        out_ref[...] = lax.dot_general(act, dn_ref[0], (((1,), (1,)), ((), ())),
                                       preferred_element_type=jnp.float32)


def _grouped(tetv, xs, up_w, down_w):
    grid_spec = pltpu.PrefetchScalarGridSpec(
        num_scalar_prefetch=1,
        grid=(NT,),
        in_specs=[
            pl.BlockSpec((TILE, H), lambda i, tetv: (i, 0)),
            pl.BlockSpec((1, 2 * DFF, H), lambda i, tetv: (tetv[0, i], 0, 0)),
            pl.BlockSpec((1, H, DFF), lambda i, tetv: (tetv[0, i], 0, 0)),
        ],
        out_specs=pl.BlockSpec((TILE, H), lambda i, tetv: (i, 0)),
    )
    return pl.pallas_call(
        _grouped_body,
        grid_spec=grid_spec,
        out_shape=jax.ShapeDtypeStruct((NPAD, H), jnp.float32),
    )(tetv, xs, up_w, down_w)


@functools.cache
def _combine_gather_kernel():
    return pl.kernel(
        _combine_gather_body,
        out_type=(
            jax.ShapeDtypeStruct((T, H), jnp.float32),
            jax.ShapeDtypeStruct((T, H), jnp.float32),
        ),
        mesh=_mesh(),
        compiler_params=pltpu.CompilerParams(needs_layout_passes=False),
        scratch_types=[
            pltpu.VMEM((TPW,), jnp.int32),
            pltpu.VMEM((TPW,), jnp.int32),
            pltpu.VMEM((CCH, H), jnp.float32),
            pltpu.SemaphoreType.DMA,
        ],
    )


def _combine_gather_body(d0_hbm, d1_hbm, ys_hbm, g0_hbm, g1_hbm,
                         d0_v, d1_v, rows_v, sem):
    base = _wid() * TPW
    pltpu.sync_copy(d0_hbm.at[pl.ds(base, TPW)], d0_v)
    pltpu.sync_copy(d1_hbm.at[pl.ds(base, TPW)], d1_v)

    def body(c, carry):
        o = c * CCH
        pltpu.async_copy(ys_hbm.at[d0_v.at[pl.ds(o, CCH)]], rows_v, sem).wait()
        pltpu.sync_copy(rows_v, g0_hbm.at[pl.ds(base + o, CCH)])
        pltpu.async_copy(ys_hbm.at[d1_v.at[pl.ds(o, CCH)]], rows_v, sem).wait()
        pltpu.sync_copy(rows_v, g1_hbm.at[pl.ds(base + o, CCH)])
        return carry
    lax.fori_loop(0, TPW // CCH, body, 0)


def _shared_body(x_ref, sup_ref, sdn_ref, out_ref):
    xb = x_ref[...]
    h = lax.dot_general(xb, sup_ref[...], (((1,), (1,)), ((), ())),
                        preferred_element_type=jnp.float32)
    g = h[:, :SDFF]
    u = h[:, SDFF:]
    act = g * jax.nn.sigmoid(g) * u
    out_ref[...] = lax.dot_general(act, sdn_ref[...], (((1,), (1,)), ((), ())),
                                   preferred_element_type=jnp.float32)


def _shared(x, sup, sdn):
    SB = 256
    return pl.pallas_call(
        _shared_body,
        grid=(T // SB,),
        in_specs=[
            pl.BlockSpec((SB, H), lambda i: (i, 0)),
            pl.BlockSpec((2 * SDFF, H), lambda i: (0, 0)),
            pl.BlockSpec((H, SDFF), lambda i: (0, 0)),
        ],
        out_specs=pl.BlockSpec((SB, H), lambda i: (i, 0)),
        out_shape=jax.ShapeDtypeStruct((T, H), jnp.float32),
    )(x, sup, sdn)


def _final_body(g0_ref, g1_ref, sh_ref, w0_ref, w1_ref, out_ref):
    out_ref[...] = (w0_ref[...] * g0_ref[...] + w1_ref[...] * g1_ref[...]
                    + sh_ref[...])


def _final(g0, g1, sh, w0, w1):
    SB = 256
    return pl.pallas_call(
        _final_body,
        grid=(T // SB,),
        in_specs=[
            pl.BlockSpec((SB, H), lambda i: (i, 0)),
            pl.BlockSpec((SB, H), lambda i: (i, 0)),
            pl.BlockSpec((SB, H), lambda i: (i, 0)),
            pl.BlockSpec((SB, 1), lambda i: (i, 0)),
            pl.BlockSpec((SB, 1), lambda i: (i, 0)),
        ],
        out_specs=pl.BlockSpec((SB, H), lambda i: (i, 0)),
        out_shape=jax.ShapeDtypeStruct((T, H), jnp.float32),
    )(g0, g1, sh, w0, w1)


def kernel(x, gate_w, up_w, down_w, shared_up_w, shared_down_w):
    d0, d1, w0, w1, tetv = _router(x, gate_w)
    d0f = d0.reshape(T)
    d1f = d1.reshape(T)
    src = _dispatch_build_kernel()(d0f, d1f)
    xs = _gather_rows_kernel()(src, x)
    sh = _shared(x, shared_up_w, shared_down_w)
    ys = _grouped(tetv, xs, up_w, down_w)
    g0, g1 = _combine_gather_kernel()(d0f, d1f, ys)
    return _final(g0, g1, sh, w0, w1)
